# baseline (speedup 1.0000x reference)
import functools

import jax
import jax.numpy as jnp
from jax import lax
from jax.experimental import pallas as pl
from jax.experimental.pallas import tpu as pltpu
from jax.experimental.pallas import tpu_sc as plsc

_S = 64.0
_L_A = 10.0
_U_A = 110.0
_L_MARGIN = 0.45
_U_MARGIN = 0.8


def _margin_body(emb_ref, cos_ref, sin_ref, loss_ref):
    emb = emb_ref[...]
    xn = jnp.sqrt(jnp.sum(emb * emb, axis=1, keepdims=True))
    xn = jnp.clip(xn, _L_A, _U_A)
    ada = (_U_MARGIN - _L_MARGIN) / (_U_A - _L_A) * (xn - _L_A) + _L_MARGIN
    cos_ref[...] = jnp.cos(ada)
    sin_ref[...] = jnp.sin(ada)
    g = xn * (1.0 / (_U_A * _U_A)) + 1.0 / xn
    loss_ref[...] = jnp.sum(g).reshape(1, 1) / emb.shape[0]


_CC = 1408


def _sc_scale(logits, B, V):
    info = plsc.get_sparse_core_info()
    nw = info.num_cores * info.num_subcores
    ngrp = B // 8
    grp_pw = ngrp // nw
    v_sc = (V // _CC) * _CC
    nch = v_sc // _CC
    total = grp_pw * nch
    mesh = plsc.VectorSubcoreMesh(core_axis_name="c", subcore_axis_name="s")

    @functools.partial(
        pl.kernel,
        out_type=jax.ShapeDtypeStruct((B, V), jnp.float32),
        mesh=mesh,
        scratch_types=[
            pltpu.VMEM((8, _CC), jnp.float32),
            pltpu.VMEM((8, _CC), jnp.float32),
            pltpu.VMEM((8, _CC), jnp.float32),
            pltpu.VMEM((8, _CC), jnp.float32),
            pltpu.SemaphoreType.DMA,
            pltpu.SemaphoreType.DMA,
            pltpu.SemaphoreType.DMA,
            pltpu.SemaphoreType.DMA,
        ],
    )
    def k(x_hbm, o_hbm, in0, in1, ou0, ou1, si0, si1, so0, so1):
        wid = lax.axis_index("s") * info.num_cores + lax.axis_index("c")
        base = wid * grp_pw * 8
        ins = (in0, in1)
        outs = (ou0, ou1)
        sis = (si0, si1)
        sos = (so0, so1)

        def src(t):
            return x_hbm.at[
                pl.ds(base + (t // nch) * 8, 8), pl.ds((t % nch) * _CC, _CC)
            ]

        def dst(t):
            return o_hbm.at[
                pl.ds(base + (t // nch) * 8, 8), pl.ds((t % nch) * _CC, _CC)
            ]

        pltpu.async_copy(src(0), in0, si0)
        pltpu.async_copy(src(1), in1, si1)

        def step(i, carry):
            for b in range(2):
                t = i * 2 + b
                pltpu.make_async_copy(src(t), ins[b], sis[b]).wait()

                @pl.when(t >= 2)
                def _():
                    pltpu.make_async_copy(outs[b], dst(t - 2), sos[b]).wait()

                for rr in range(8):

                    def mul(kk, c, b=b, rr=rr):
                        for u in range(8):
                            o = kk * 128 + u * 16
                            x16 = ins[b][rr, pl.ds(o, 16)]
                            outs[b][rr, pl.ds(o, 16)] = x16 * _S
                        return c

                    lax.fori_loop(0, _CC // 128, mul, 0)
                pltpu.async_copy(outs[b], dst(t), sos[b])

                @pl.when(t + 2 < total)
                def _():
                    pltpu.async_copy(src(t + 2), ins[b], sis[b])

            return carry

        lax.fori_loop(0, total // 2, step, 0)
        pltpu.make_async_copy(ou0, dst(total - 2), so0).wait()
        pltpu.make_async_copy(ou1, dst(total - 1), so1).wait()

    return k(logits)


def _stripe_body(alias_ref, x_ref, o_ref):
    del alias_ref
    o_ref[...] = x_ref[...] * _S


def _patch_body(lab_ref, x_ref, cos_ref, sin_ref, o_ref):
    s = pl.program_id(0)
    r = (s % 128) * 8 + s // 128
    lab = lab_ref[r]
    col0 = (lab // 128) * 128
    w = x_ref[...]
    rowm = lax.broadcasted_iota(jnp.int32, (8, 128), 0) == s // 128
    colm = lax.broadcasted_iota(jnp.int32, (8, 128), 1) + col0 == lab
    m = rowm & colm
    t = jnp.sum(jnp.where(m, w, 0.0)).reshape(1, 1) * (1.0 / _S)
    sin_t = jnp.sqrt(jnp.maximum(1.0 - t * t, 0.0))
    nv = (t * cos_ref[r] - sin_t * sin_ref[r]) * _S
    o_ref[...] = jnp.where(m, nv, w)


def kernel(logits, labels, embeddings):
    B, V = logits.shape
    labels = labels.astype(jnp.int32)

    cos_m, sin_m, loss = pl.pallas_call(
        _margin_body,
        out_shape=(
            jax.ShapeDtypeStruct((B, 1), jnp.float32),
            jax.ShapeDtypeStruct((B, 1), jnp.float32),
            jax.ShapeDtypeStruct((1, 1), jnp.float32),
        ),
        in_specs=[pl.BlockSpec(embeddings.shape, lambda: (0, 0))],
        out_specs=(
            pl.BlockSpec((B, 1), lambda: (0, 0)),
            pl.BlockSpec((B, 1), lambda: (0, 0)),
            pl.BlockSpec((1, 1), lambda: (0, 0)),
        ),
    )(embeddings)

    scaled = _sc_scale(logits, B, V)

    v_sc = (V // _CC) * _CC
    if v_sc < V:
        jtile = v_sc // 128
        scaled = pl.pallas_call(
            _stripe_body,
            grid=(B // 8,),
            in_specs=[
                pl.BlockSpec(memory_space=pl.ANY),
                pl.BlockSpec((8, 128), lambda i: (i, jtile)),
            ],
            out_specs=pl.BlockSpec((8, 128), lambda i: (i, jtile)),
            out_shape=jax.ShapeDtypeStruct((B, V), jnp.float32),
            input_output_aliases={0: 0},
        )(scaled, logits)

    def _win_idx(s, lab):
        r = (s % 128) * 8 + s // 128
        return (s % 128, lab[r] // 128)

    grid_spec = pltpu.PrefetchScalarGridSpec(
        num_scalar_prefetch=1,
        grid=(B,),
        in_specs=[
            pl.BlockSpec((8, 128), _win_idx),
            pl.BlockSpec(memory_space=pltpu.SMEM),
            pl.BlockSpec(memory_space=pltpu.SMEM),
        ],
        out_specs=pl.BlockSpec((8, 128), _win_idx),
    )
    out = pl.pallas_call(
        _patch_body,
        grid_spec=grid_spec,
        out_shape=jax.ShapeDtypeStruct((B, V), jnp.float32),
        input_output_aliases={1: 0},
    )(labels, scaled, cos_m.reshape(B), sin_m.reshape(B))

    return (out, loss.reshape(()))
